# Initial kernel scaffold; baseline (speedup 1.0000x reference)
import jax
import jax.numpy as jnp
from jax.experimental import pallas as pl


def kernel(input, W, b):
    raise NotImplementedError("write your pallas kernel here")


# speedup vs baseline: 2.0985x; 2.0985x over previous
import jax
import jax.numpy as jnp
from jax.experimental import pallas as pl
from jax.experimental.pallas import tpu as pltpu

_DIM = 768
_NUM_OUT = 64
_N_TOK = 32768
_BM = 1024

_NEG_INF = float("-inf")


def _router_block(x_ref, w_ref, b_ref, probs_ref, idx_ref):
    x = x_ref[...]
    w = w_ref[...]
    logits = jax.lax.dot_general(
        x, w, (((1,), (1,)), ((), ())), preferred_element_type=jnp.float32
    )
    logits = logits + b_ref[...]

    iota = jax.lax.broadcasted_iota(jnp.int32, logits.shape, 1)
    big = jnp.int32(_NUM_OUT)

    v1 = jnp.max(logits, axis=-1, keepdims=True)
    i1 = jnp.min(jnp.where(logits == v1, iota, big), axis=-1, keepdims=True)
    masked = jnp.where(iota == i1, _NEG_INF, logits)
    v2 = jnp.max(masked, axis=-1, keepdims=True)
    i2 = jnp.min(jnp.where(masked == v2, iota, big), axis=-1, keepdims=True)

    t = jnp.exp(v2 - v1)
    denom = 1.0 + t
    p1 = 1.0 / denom
    p2 = t / denom

    probs_ref[...] = jnp.concatenate([p1, p2], axis=-1)
    idx_ref[...] = jnp.concatenate([i1, i2], axis=-1)


def kernel(input, W, b):
    n_tok = input.shape[0]
    grid = (n_tok // _BM,)
    b2d = b.reshape(1, _NUM_OUT)
    probs, idx = pl.pallas_call(
        _router_block,
        grid=grid,
        in_specs=[
            pl.BlockSpec((_BM, _DIM), lambda i: (i, 0)),
            pl.BlockSpec((_NUM_OUT, _DIM), lambda i: (0, 0)),
            pl.BlockSpec((1, _NUM_OUT), lambda i: (0, 0)),
        ],
        out_specs=[
            pl.BlockSpec((_BM, 2), lambda i: (i, 0)),
            pl.BlockSpec((_BM, 2), lambda i: (i, 0)),
        ],
        out_shape=[
            jax.ShapeDtypeStruct((n_tok, 2), jnp.float32),
            jax.ShapeDtypeStruct((n_tok, 2), jnp.int32),
        ],
        compiler_params=pltpu.CompilerParams(
            dimension_semantics=("arbitrary",),
        ),
    )(input, W, b2d)
    return probs, idx


# speedup vs baseline: 2.2452x; 1.0699x over previous
import jax
import jax.numpy as jnp
from jax.experimental import pallas as pl
from jax.experimental.pallas import tpu as pltpu

_DIM = 768
_NUM_OUT = 64
_N_TOK = 32768
_BM = 1024

_NEG_INF = float("-inf")


def _router_block(x_ref, w_ref, b_ref, probs_ref, idx_ref):
    x = x_ref[...]
    w = w_ref[...]
    logits = jax.lax.dot_general(
        x, w, (((1,), (1,)), ((), ())), preferred_element_type=jnp.float32
    )
    logits = logits + b_ref[...]

    iota = jax.lax.broadcasted_iota(jnp.int32, logits.shape, 1).astype(jnp.float32)
    big = float(_NUM_OUT)

    v1 = jnp.max(logits, axis=-1, keepdims=True)
    i1f = jnp.min(jnp.where(logits == v1, iota, big), axis=-1, keepdims=True)
    masked = jnp.where(iota == i1f, _NEG_INF, logits)
    v2 = jnp.max(masked, axis=-1, keepdims=True)
    i2f = jnp.min(jnp.where(masked == v2, iota, big), axis=-1, keepdims=True)
    i1 = i1f.astype(jnp.int32)
    i2 = i2f.astype(jnp.int32)

    t = jnp.exp(v2 - v1)
    denom = 1.0 + t
    p1 = 1.0 / denom
    p2 = t / denom

    probs_ref[...] = jnp.concatenate([p1, p2], axis=-1)
    idx_ref[...] = jnp.concatenate([i1, i2], axis=-1)


def kernel(input, W, b):
    n_tok = input.shape[0]
    grid = (n_tok // _BM,)
    b2d = b.reshape(1, _NUM_OUT)
    probs, idx = pl.pallas_call(
        _router_block,
        grid=grid,
        in_specs=[
            pl.BlockSpec((_BM, _DIM), lambda i: (i, 0)),
            pl.BlockSpec((_NUM_OUT, _DIM), lambda i: (0, 0)),
            pl.BlockSpec((1, _NUM_OUT), lambda i: (0, 0)),
        ],
        out_specs=[
            pl.BlockSpec((_BM, 2), lambda i: (i, 0)),
            pl.BlockSpec((_BM, 2), lambda i: (i, 0)),
        ],
        out_shape=[
            jax.ShapeDtypeStruct((n_tok, 2), jnp.float32),
            jax.ShapeDtypeStruct((n_tok, 2), jnp.int32),
        ],
        compiler_params=pltpu.CompilerParams(
            dimension_semantics=("arbitrary",),
        ),
    )(input, W, b2d)
    return probs, idx


# speedup vs baseline: 2.5407x; 1.1316x over previous
import jax
import jax.numpy as jnp
from jax.experimental import pallas as pl
from jax.experimental.pallas import tpu as pltpu

_DIM = 768
_NUM_OUT = 64
_N_TOK = 32768
_BM = 2048

_NEG_INF = float("-inf")


def _router_block(x_ref, w_ref, b_ref, probs_ref, idx_ref):
    x = x_ref[...]
    w = w_ref[...]
    logits = jax.lax.dot_general(
        x, w, (((1,), (1,)), ((), ())), preferred_element_type=jnp.float32
    )
    logits = logits + b_ref[...]

    iota = jax.lax.broadcasted_iota(jnp.int32, logits.shape, 1).astype(jnp.float32)
    big = float(_NUM_OUT)

    v1 = jnp.max(logits, axis=-1, keepdims=True)
    i1f = jnp.min(jnp.where(logits == v1, iota, big), axis=-1, keepdims=True)
    masked = jnp.where(iota == i1f, _NEG_INF, logits)
    v2 = jnp.max(masked, axis=-1, keepdims=True)
    i2f = jnp.min(jnp.where(masked == v2, iota, big), axis=-1, keepdims=True)
    i1 = i1f.astype(jnp.int32)
    i2 = i2f.astype(jnp.int32)

    t = jnp.exp(v2 - v1)
    denom = 1.0 + t
    p1 = 1.0 / denom
    p2 = t / denom

    probs_ref[...] = jnp.concatenate([p1, p2], axis=-1)
    idx_ref[...] = jnp.concatenate([i1, i2], axis=-1)


def kernel(input, W, b):
    n_tok = input.shape[0]
    grid = (n_tok // _BM,)
    b2d = b.reshape(1, _NUM_OUT)
    probs, idx = pl.pallas_call(
        _router_block,
        grid=grid,
        in_specs=[
            pl.BlockSpec((_BM, _DIM), lambda i: (i, 0)),
            pl.BlockSpec((_NUM_OUT, _DIM), lambda i: (0, 0)),
            pl.BlockSpec((1, _NUM_OUT), lambda i: (0, 0)),
        ],
        out_specs=[
            pl.BlockSpec((_BM, 2), lambda i: (i, 0)),
            pl.BlockSpec((_BM, 2), lambda i: (i, 0)),
        ],
        out_shape=[
            jax.ShapeDtypeStruct((n_tok, 2), jnp.float32),
            jax.ShapeDtypeStruct((n_tok, 2), jnp.int32),
        ],
        compiler_params=pltpu.CompilerParams(
            dimension_semantics=("arbitrary",),
        ),
    )(input, W, b2d)
    return probs, idx


# speedup vs baseline: 2.6696x; 1.0507x over previous
import jax
import jax.numpy as jnp
from jax.experimental import pallas as pl
from jax.experimental.pallas import tpu as pltpu

_DIM = 768
_NUM_OUT = 64
_N_TOK = 32768
_BM = 4096

_NEG_INF = float("-inf")


def _router_block(x_ref, w_ref, b_ref, probs_ref, idx_ref):
    x = x_ref[...]
    w = w_ref[...]
    logits = jax.lax.dot_general(
        x, w, (((1,), (1,)), ((), ())), preferred_element_type=jnp.float32
    )
    logits = logits + b_ref[...]

    iota = jax.lax.broadcasted_iota(jnp.int32, logits.shape, 1).astype(jnp.float32)
    big = float(_NUM_OUT)

    v1 = jnp.max(logits, axis=-1, keepdims=True)
    i1f = jnp.min(jnp.where(logits == v1, iota, big), axis=-1, keepdims=True)
    masked = jnp.where(iota == i1f, _NEG_INF, logits)
    v2 = jnp.max(masked, axis=-1, keepdims=True)
    i2f = jnp.min(jnp.where(masked == v2, iota, big), axis=-1, keepdims=True)
    i1 = i1f.astype(jnp.int32)
    i2 = i2f.astype(jnp.int32)

    t = jnp.exp(v2 - v1)
    denom = 1.0 + t
    p1 = 1.0 / denom
    p2 = t / denom

    probs_ref[...] = jnp.concatenate([p1, p2], axis=-1)
    idx_ref[...] = jnp.concatenate([i1, i2], axis=-1)


def kernel(input, W, b):
    n_tok = input.shape[0]
    grid = (n_tok // _BM,)
    b2d = b.reshape(1, _NUM_OUT)
    probs, idx = pl.pallas_call(
        _router_block,
        grid=grid,
        in_specs=[
            pl.BlockSpec((_BM, _DIM), lambda i: (i, 0)),
            pl.BlockSpec((_NUM_OUT, _DIM), lambda i: (0, 0)),
            pl.BlockSpec((1, _NUM_OUT), lambda i: (0, 0)),
        ],
        out_specs=[
            pl.BlockSpec((_BM, 2), lambda i: (i, 0)),
            pl.BlockSpec((_BM, 2), lambda i: (i, 0)),
        ],
        out_shape=[
            jax.ShapeDtypeStruct((n_tok, 2), jnp.float32),
            jax.ShapeDtypeStruct((n_tok, 2), jnp.int32),
        ],
        compiler_params=pltpu.CompilerParams(
            dimension_semantics=("arbitrary",),
        ),
    )(input, W, b2d)
    return probs, idx
